# speedup vs baseline: 1.6455x; 1.6455x over previous
import jax
import jax.numpy as jnp
from jax.experimental import pallas as pl

_N = 10000
_E = 320000
_IN = 128
_HGL = 64
_HID = 128
_OUT = 128


def _mm_kernel(x_ref, wgl_ref, w1_ref, h_ref, x1_ref):
    x = x_ref[...]
    h_ref[...] = x @ wgl_ref[...]
    x1_ref[...] = x @ w1_ref[...]


def kernel(inputs, edge, W_gl, a_gl, W1, W2):
    src = edge[0]
    dst = edge[1]

    h, x1 = pl.pallas_call(
        _mm_kernel,
        out_shape=[
            jax.ShapeDtypeStruct((_N, _HGL), jnp.float32),
            jax.ShapeDtypeStruct((_N, _HID), jnp.float32),
        ],
    )(inputs, W_gl, W1)

    diff = jnp.abs(h[src] - h[dst])
    e = jax.nn.relu(diff @ a_gl)
    ex = jnp.exp(e)
    s = jax.ops.segment_sum(ex, dst, num_segments=_N)
    r = 1.0 / (s + 1e-16)
    adj = ex * r[dst]

    y1 = jax.ops.segment_sum(ex[:, None] * x1[src], dst, num_segments=_N)
    y1 = jax.nn.relu(y1 * r[:, None])

    x2 = y1 @ W2
    y2 = jax.ops.segment_sum(ex[:, None] * x2[src], dst, num_segments=_N)
    y2 = y2 * r[:, None]

    return (y2, h, adj)


# speedup vs baseline: 5.5549x; 3.3758x over previous
import jax
import jax.numpy as jnp
from jax import lax
from jax.experimental import pallas as pl
from jax.experimental.pallas import tpu as pltpu
from jax.experimental.pallas import tpu_sc as plsc

_N = 10000
_E = 320000
_HGL = 64
_HID = 128
_NC = 2
_NS = 16
_EPT = _E // (_NC * _NS)
_CH = 80
_NCHUNK = _EPT // _CH
_RPT = 624
_TAIL = _N - _NS * _RPT

_f32 = jnp.float32
_i32 = jnp.int32


def _mm_kernel(x_ref, wgl_ref, w1_ref, h_ref, x1_ref):
    x = x_ref[...]
    h_ref[...] = x @ wgl_ref[...]
    x1_ref[...] = x @ w1_ref[...]


def _mid_kernel(yun_ref, r_ref, w2_ref, x2_ref):
    y = yun_ref[0] + yun_ref[1]
    y = jnp.maximum(y * r_ref[...], 0.0)
    x2_ref[...] = y @ w2_ref[...]


def _fin_kernel(yun_ref, r_ref, out_ref):
    out_ref[...] = (yun_ref[0] + yun_ref[1]) * r_ref[...]


def _scale_rows(xs, exc):
    def gbody(g, carry):
        ev = exc[pl.ds(g * 16, 16)]
        for j in range(16):
            e = g * 16 + j
            w = ev[j]
            for k in range(_HID // 16):
                xs[e, pl.ds(k * 16, 16)] = xs[e, pl.ds(k * 16, 16)] * w
        return carry

    lax.fori_loop(0, _CH // 16, gbody, 0)


def _acc_init(z_hbm, z1_hbm, y_acc, s_acc, sb, s):
    pltpu.sync_copy(z_hbm.at[pl.ds(s * _RPT, _RPT), :],
                    y_acc.at[pl.ds(s * _RPT, _RPT), :])
    pltpu.sync_copy(z1_hbm.at[pl.ds(s * _RPT, _RPT)], sb)
    pltpu.sync_copy(sb, s_acc.at[pl.ds(s * _RPT, _RPT)])

    @pl.when(s == _NS - 1)
    def _():
        pltpu.sync_copy(z_hbm.at[pl.ds(_NS * _RPT, _TAIL), :],
                        y_acc.at[pl.ds(_NS * _RPT, _TAIL), :])
        pltpu.sync_copy(sb.at[pl.ds(0, _TAIL)],
                        s_acc.at[pl.ds(_NS * _RPT, _TAIL)])

    plsc.subcore_barrier()


def _edge_kernel(hp_hbm, x_hbm, src_hbm, dst_hbm, a_hbm, z_hbm, z1_hbm,
                 yun_hbm, s01_hbm, ex_hbm,
                 sidx, didx, s2idx, d2idx, hs, hd, xs, exc, a_buf, sb,
                 y_acc, s_acc):
    c = lax.axis_index("c")
    s = lax.axis_index("s")
    base = (c * _NS + s) * _EPT

    pltpu.sync_copy(a_hbm, a_buf)
    _acc_init(z_hbm, z1_hbm, y_acc, s_acc, sb, s)

    iota = lax.iota(_i32, 16)

    def chunk(i, carry):
        off = base + i * _CH
        pltpu.sync_copy(src_hbm.at[pl.ds(off, _CH)], sidx)
        pltpu.sync_copy(dst_hbm.at[pl.ds(off, _CH)], didx)
        for g in range(_CH // 16):
            s16 = sidx[pl.ds(g * 16, 16)]
            d16 = didx[pl.ds(g * 16, 16)]
            s2idx[pl.ds(g * 16, 16)] = jnp.right_shift(s16, 1)
            d2idx[pl.ds(g * 16, 16)] = jnp.right_shift(d16, 1)
        pltpu.sync_copy(hp_hbm.at[s2idx], hs)
        pltpu.sync_copy(hp_hbm.at[d2idx], hd)
        pltpu.sync_copy(x_hbm.at[sidx], xs)

        for g in range(_CH // 16):
            s16 = sidx[pl.ds(g * 16, 16)]
            d16 = didx[pl.ds(g * 16, 16)]
            ps = (s16 & 1) * _HGL
            pd = (d16 & 1) * _HGL
            rows = g * 16 + iota
            acc = jnp.zeros((16,), _f32)
            for fb in range(_HGL // 16):
                av = a_buf[pl.ds(fb * 16, 16)]
                for j in range(16):
                    f = fb * 16 + j
                    hsv = plsc.load_gather(hs, [rows, ps + f])
                    hdv = plsc.load_gather(hd, [rows, pd + f])
                    acc = acc + jnp.abs(hsv - hdv) * av[j]
            exc[pl.ds(g * 16, 16)] = jnp.exp(jnp.maximum(acc, 0.0))

        _scale_rows(xs, exc)
        pltpu.sync_copy(exc, ex_hbm.at[pl.ds(off, _CH)])
        pltpu.sync_copy(xs, y_acc.at[didx], add=True)
        pltpu.sync_copy(exc, s_acc.at[didx], add=True)
        return carry

    lax.fori_loop(0, _NCHUNK, chunk, 0)
    plsc.subcore_barrier()

    pltpu.sync_copy(y_acc.at[pl.ds(s * _RPT, _RPT), :],
                    yun_hbm.at[c, pl.ds(s * _RPT, _RPT), :])
    pltpu.sync_copy(s_acc.at[pl.ds(s * _RPT, _RPT)], sb)
    pltpu.sync_copy(sb, s01_hbm.at[pl.ds(c * _N + s * _RPT, _RPT)])

    @pl.when(s == _NS - 1)
    def _():
        pltpu.sync_copy(y_acc.at[pl.ds(_NS * _RPT, _TAIL), :],
                        yun_hbm.at[c, pl.ds(_NS * _RPT, _TAIL), :])
        pltpu.sync_copy(s_acc.at[pl.ds(_NS * _RPT, _TAIL)],
                        sb.at[pl.ds(0, _TAIL)])
        pltpu.sync_copy(sb.at[pl.ds(0, _TAIL)],
                        s01_hbm.at[pl.ds(c * _N + _NS * _RPT, _TAIL)])


def _spmm_kernel(x_hbm, src_hbm, dst_hbm, exg_hbm, r_hbm, z_hbm,
                 yun_hbm, adj_hbm,
                 sidx, didx, xs, exc, adjc, r_vmem, y_acc):
    c = lax.axis_index("c")
    s = lax.axis_index("s")
    base = (c * _NS + s) * _EPT

    pltpu.sync_copy(r_hbm, r_vmem)
    pltpu.sync_copy(z_hbm.at[pl.ds(s * _RPT, _RPT), :],
                    y_acc.at[pl.ds(s * _RPT, _RPT), :])

    @pl.when(s == _NS - 1)
    def _():
        pltpu.sync_copy(z_hbm.at[pl.ds(_NS * _RPT, _TAIL), :],
                        y_acc.at[pl.ds(_NS * _RPT, _TAIL), :])

    plsc.subcore_barrier()

    def chunk(i, carry):
        off = base + i * _CH
        pltpu.sync_copy(src_hbm.at[pl.ds(off, _CH)], sidx)
        pltpu.sync_copy(dst_hbm.at[pl.ds(off, _CH)], didx)
        pltpu.sync_copy(exg_hbm.at[pl.ds(off, _CH)], exc)
        pltpu.sync_copy(x_hbm.at[sidx], xs)

        for g in range(_CH // 16):
            d16 = didx[pl.ds(g * 16, 16)]
            rv = plsc.load_gather(r_vmem, [d16])
            adjc[pl.ds(g * 16, 16)] = exc[pl.ds(g * 16, 16)] * rv

        _scale_rows(xs, exc)
        pltpu.sync_copy(adjc, adj_hbm.at[pl.ds(off, _CH)])
        pltpu.sync_copy(xs, y_acc.at[didx], add=True)
        return carry

    lax.fori_loop(0, _NCHUNK, chunk, 0)
    plsc.subcore_barrier()

    pltpu.sync_copy(y_acc.at[pl.ds(s * _RPT, _RPT), :],
                    yun_hbm.at[c, pl.ds(s * _RPT, _RPT), :])

    @pl.when(s == _NS - 1)
    def _():
        pltpu.sync_copy(y_acc.at[pl.ds(_NS * _RPT, _TAIL), :],
                        yun_hbm.at[c, pl.ds(_NS * _RPT, _TAIL), :])


def kernel(inputs, edge, W_gl, a_gl, W1, W2):
    src = edge[0].astype(_i32)
    dst = edge[1].astype(_i32)
    zeros = jnp.zeros((_N, _HID), _f32)
    zeros1 = jnp.zeros((_N,), _f32)

    h, x1 = pl.pallas_call(
        _mm_kernel,
        out_shape=[
            jax.ShapeDtypeStruct((_N, _HGL), _f32),
            jax.ShapeDtypeStruct((_N, _HID), _f32),
        ],
    )(inputs, W_gl, W1)
    hp = h.reshape(_N // 2, 2 * _HGL)

    edge_k = pl.kernel(
        _edge_kernel,
        out_type=[
            jax.ShapeDtypeStruct((_NC, _N, _HID), _f32),
            jax.ShapeDtypeStruct((_NC * _N,), _f32),
            jax.ShapeDtypeStruct((_E,), _f32),
        ],
        mesh=plsc.VectorSubcoreMesh(core_axis_name="c", subcore_axis_name="s"),
        compiler_params=pltpu.CompilerParams(needs_layout_passes=False),
        scratch_types=[
            pltpu.VMEM((_CH,), _i32),
            pltpu.VMEM((_CH,), _i32),
            pltpu.VMEM((_CH,), _i32),
            pltpu.VMEM((_CH,), _i32),
            pltpu.VMEM((_CH, 2 * _HGL), _f32),
            pltpu.VMEM((_CH, 2 * _HGL), _f32),
            pltpu.VMEM((_CH, _HID), _f32),
            pltpu.VMEM((_CH,), _f32),
            pltpu.VMEM((_HGL,), _f32),
            pltpu.VMEM((_RPT,), _f32),
            pltpu.VMEM_SHARED((_N, _HID), _f32),
            pltpu.VMEM_SHARED((_N,), _f32),
        ],
    )
    y1un, s01, ex = edge_k(hp, x1, src, dst, a_gl, zeros, zeros1)

    s = s01[:_N] + s01[_N:]
    r = 1.0 / (s + 1e-16)
    r2d = r[:, None]

    x2 = pl.pallas_call(
        _mid_kernel,
        out_shape=jax.ShapeDtypeStruct((_N, _HID), _f32),
    )(y1un, r2d, W2)

    spmm_k = pl.kernel(
        _spmm_kernel,
        out_type=[
            jax.ShapeDtypeStruct((_NC, _N, _HID), _f32),
            jax.ShapeDtypeStruct((_E,), _f32),
        ],
        mesh=plsc.VectorSubcoreMesh(core_axis_name="c", subcore_axis_name="s"),
        compiler_params=pltpu.CompilerParams(needs_layout_passes=False),
        scratch_types=[
            pltpu.VMEM((_CH,), _i32),
            pltpu.VMEM((_CH,), _i32),
            pltpu.VMEM((_CH, _HID), _f32),
            pltpu.VMEM((_CH,), _f32),
            pltpu.VMEM((_CH,), _f32),
            pltpu.VMEM((_N,), _f32),
            pltpu.VMEM_SHARED((_N, _HID), _f32),
        ],
    )
    y2un, adj = spmm_k(x2, src, dst, ex, r, zeros)

    y2 = pl.pallas_call(
        _fin_kernel,
        out_shape=jax.ShapeDtypeStruct((_N, _HID), _f32),
    )(y2un, r2d)

    return (y2, h, adj)


# speedup vs baseline: 5.5620x; 1.0013x over previous
import jax
import jax.numpy as jnp
from jax import lax
from jax.experimental import pallas as pl
from jax.experimental.pallas import tpu as pltpu
from jax.experimental.pallas import tpu_sc as plsc

_N = 10000
_E = 320000
_HGL = 64
_HID = 128
_NC = 2
_NS = 16
_EPT = _E // (_NC * _NS)
_CH = 80
_NCHUNK = _EPT // _CH
_RPT = 624
_TAIL = _N - _NS * _RPT

_f32 = jnp.float32
_i32 = jnp.int32


def _mm_kernel(x_ref, wgl_ref, w1_ref, h_ref, x1_ref):
    x = x_ref[...]
    h_ref[...] = x @ wgl_ref[...]
    x1_ref[...] = x @ w1_ref[...]


def _mid_kernel(yun_ref, r_ref, w2_ref, x2_ref):
    y = yun_ref[0] + yun_ref[1]
    y = jnp.maximum(y * r_ref[...], 0.0)
    x2_ref[...] = y @ w2_ref[...]


def _fin_kernel(yun_ref, r_ref, out_ref):
    out_ref[...] = (yun_ref[0] + yun_ref[1]) * r_ref[...]


def _scale_rows(xs, exc):
    def gbody(g, carry):
        ev = exc[pl.ds(g * 16, 16)]
        for j in range(16):
            e = g * 16 + j
            w = ev[j]
            for k in range(_HID // 16):
                xs[e, pl.ds(k * 16, 16)] = xs[e, pl.ds(k * 16, 16)] * w
        return carry

    lax.fori_loop(0, _CH // 16, gbody, 0)


def _acc_init(z_hbm, z1_hbm, y_acc, s_acc, sb, s):
    pltpu.sync_copy(z_hbm.at[pl.ds(s * _RPT, _RPT), :],
                    y_acc.at[pl.ds(s * _RPT, _RPT), :])
    pltpu.sync_copy(z1_hbm.at[pl.ds(s * _RPT, _RPT)], sb)
    pltpu.sync_copy(sb, s_acc.at[pl.ds(s * _RPT, _RPT)])

    @pl.when(s == _NS - 1)
    def _():
        pltpu.sync_copy(z_hbm.at[pl.ds(_NS * _RPT, _TAIL), :],
                        y_acc.at[pl.ds(_NS * _RPT, _TAIL), :])
        pltpu.sync_copy(sb.at[pl.ds(0, _TAIL)],
                        s_acc.at[pl.ds(_NS * _RPT, _TAIL)])

    plsc.subcore_barrier()


def _edge_kernel(hp_hbm, x_hbm, src_hbm, dst_hbm, a_hbm, z_hbm, z1_hbm,
                 yun_hbm, s01_hbm, ex_hbm,
                 sidx, didx, s2, d2, xs,
                 hs, hd, exc, a_buf, sb, y_acc, s_acc,
                 sem, semw):
    c = lax.axis_index("c")
    s = lax.axis_index("s")
    base = (c * _NS + s) * _EPT

    pltpu.sync_copy(a_hbm, a_buf)
    _acc_init(z_hbm, z1_hbm, y_acc, s_acc, sb, s)

    iota = lax.iota(_i32, 16)

    def score():
        def gbody(g, carry):
            s16 = sidx[pl.ds(g * 16, 16)]
            d16 = didx[pl.ds(g * 16, 16)]
            ps = (s16 & 1) * _HGL
            pd = (d16 & 1) * _HGL
            rows = g * 16 + iota

            def fbody(fb, acc):
                av = a_buf[pl.ds(fb * 16, 16)]
                for j in range(16):
                    cs = ps + (fb * 16 + j)
                    cd = pd + (fb * 16 + j)
                    hsv = plsc.load_gather(hs, [rows, cs])
                    hdv = plsc.load_gather(hd, [rows, cd])
                    acc = acc + jnp.abs(hsv - hdv) * av[j]
                return acc

            acc = lax.fori_loop(0, _HGL // 16, fbody, jnp.zeros((16,), _f32))
            exc[pl.ds(g * 16, 16)] = jnp.exp(jnp.maximum(acc, 0.0))
            return carry

        lax.fori_loop(0, _CH // 16, gbody, 0)

    def step(j, carry):
        off = base + j * _CH
        pltpu.sync_copy(src_hbm.at[pl.ds(off, _CH)], sidx)
        pltpu.sync_copy(dst_hbm.at[pl.ds(off, _CH)], didx)

        def gb(g, carry2):
            s16 = sidx[pl.ds(g * 16, 16)]
            d16 = didx[pl.ds(g * 16, 16)]
            s2[pl.ds(g * 16, 16)] = jnp.right_shift(s16, 1)
            d2[pl.ds(g * 16, 16)] = jnp.right_shift(d16, 1)
            return carry2

        lax.fori_loop(0, _CH // 16, gb, 0)

        w = pltpu.async_copy(x_hbm.at[sidx], xs, sem)
        w.wait()
        w = pltpu.async_copy(hp_hbm.at[s2], hs, sem)
        w.wait()
        w = pltpu.async_copy(hp_hbm.at[d2], hd, sem)
        w.wait()

        score()
        _scale_rows(xs, exc)

        w = pltpu.async_copy(exc, ex_hbm.at[pl.ds(off, _CH)], semw)
        w.wait()
        w = pltpu.async_copy(xs, y_acc.at[didx], semw, add=True)
        w.wait()
        w = pltpu.async_copy(exc, s_acc.at[didx], semw, add=True)
        w.wait()
        return carry

    lax.fori_loop(0, _NCHUNK, step, 0)

    plsc.subcore_barrier()

    pltpu.sync_copy(y_acc.at[pl.ds(s * _RPT, _RPT), :],
                    yun_hbm.at[c, pl.ds(s * _RPT, _RPT), :])
    pltpu.sync_copy(s_acc.at[pl.ds(s * _RPT, _RPT)], sb)
    pltpu.sync_copy(sb, s01_hbm.at[pl.ds(c * _N + s * _RPT, _RPT)])

    @pl.when(s == _NS - 1)
    def _():
        pltpu.sync_copy(y_acc.at[pl.ds(_NS * _RPT, _TAIL), :],
                        yun_hbm.at[c, pl.ds(_NS * _RPT, _TAIL), :])
        pltpu.sync_copy(s_acc.at[pl.ds(_NS * _RPT, _TAIL)],
                        sb.at[pl.ds(0, _TAIL)])
        pltpu.sync_copy(sb.at[pl.ds(0, _TAIL)],
                        s01_hbm.at[pl.ds(c * _N + _NS * _RPT, _TAIL)])


def _spmm_kernel(x_hbm, src_hbm, dst_hbm, exg_hbm, r_hbm, z_hbm,
                 yun_hbm, adj_hbm,
                 sidx, didx, xs, exc,
                 adjc, r_vmem, y_acc, sem, semw):
    c = lax.axis_index("c")
    s = lax.axis_index("s")
    base = (c * _NS + s) * _EPT

    pltpu.sync_copy(r_hbm, r_vmem)
    pltpu.sync_copy(z_hbm.at[pl.ds(s * _RPT, _RPT), :],
                    y_acc.at[pl.ds(s * _RPT, _RPT), :])

    @pl.when(s == _NS - 1)
    def _():
        pltpu.sync_copy(z_hbm.at[pl.ds(_NS * _RPT, _TAIL), :],
                        y_acc.at[pl.ds(_NS * _RPT, _TAIL), :])

    plsc.subcore_barrier()

    def step(j, carry):
        off = base + j * _CH
        pltpu.sync_copy(src_hbm.at[pl.ds(off, _CH)], sidx)
        pltpu.sync_copy(dst_hbm.at[pl.ds(off, _CH)], didx)
        w = pltpu.async_copy(exg_hbm.at[pl.ds(off, _CH)], exc, sem)
        w.wait()
        w = pltpu.async_copy(x_hbm.at[sidx], xs, sem)
        w.wait()

        def gbody(g, carry2):
            d16 = didx[pl.ds(g * 16, 16)]
            rv = plsc.load_gather(r_vmem, [d16])
            adjc[pl.ds(g * 16, 16)] = exc[pl.ds(g * 16, 16)] * rv
            return carry2

        lax.fori_loop(0, _CH // 16, gbody, 0)
        _scale_rows(xs, exc)
        w = pltpu.async_copy(adjc, adj_hbm.at[pl.ds(off, _CH)], semw)
        w.wait()
        w = pltpu.async_copy(xs, y_acc.at[didx], semw, add=True)
        w.wait()
        return carry

    lax.fori_loop(0, _NCHUNK, step, 0)

    plsc.subcore_barrier()

    pltpu.sync_copy(y_acc.at[pl.ds(s * _RPT, _RPT), :],
                    yun_hbm.at[c, pl.ds(s * _RPT, _RPT), :])

    @pl.when(s == _NS - 1)
    def _():
        pltpu.sync_copy(y_acc.at[pl.ds(_NS * _RPT, _TAIL), :],
                        yun_hbm.at[c, pl.ds(_NS * _RPT, _TAIL), :])


def kernel(inputs, edge, W_gl, a_gl, W1, W2):
    src = edge[0].astype(_i32)
    dst = edge[1].astype(_i32)
    zeros = jnp.zeros((_N, _HID), _f32)
    zeros1 = jnp.zeros((_N,), _f32)

    h, x1 = pl.pallas_call(
        _mm_kernel,
        out_shape=[
            jax.ShapeDtypeStruct((_N, _HGL), _f32),
            jax.ShapeDtypeStruct((_N, _HID), _f32),
        ],
    )(inputs, W_gl, W1)
    hp = h.reshape(_N // 2, 2 * _HGL)

    edge_k = pl.kernel(
        _edge_kernel,
        out_type=[
            jax.ShapeDtypeStruct((_NC, _N, _HID), _f32),
            jax.ShapeDtypeStruct((_NC * _N,), _f32),
            jax.ShapeDtypeStruct((_E,), _f32),
        ],
        mesh=plsc.VectorSubcoreMesh(core_axis_name="c", subcore_axis_name="s"),
        compiler_params=pltpu.CompilerParams(needs_layout_passes=False),
        scratch_types=(
            [
                pltpu.VMEM((_CH,), _i32),
                pltpu.VMEM((_CH,), _i32),
                pltpu.VMEM((_CH,), _i32),
                pltpu.VMEM((_CH,), _i32),
                pltpu.VMEM((_CH, _HID), _f32),
                pltpu.VMEM((_CH, 2 * _HGL), _f32),
                pltpu.VMEM((_CH, 2 * _HGL), _f32),
                pltpu.VMEM((_CH,), _f32),
                pltpu.VMEM((_HGL,), _f32),
                pltpu.VMEM((_RPT,), _f32),
                pltpu.VMEM_SHARED((_N, _HID), _f32),
                pltpu.VMEM_SHARED((_N,), _f32),
                pltpu.SemaphoreType.DMA,
                pltpu.SemaphoreType.DMA,
            ]
        ),
    )
    y1un, s01, ex = edge_k(hp, x1, src, dst, a_gl, zeros, zeros1)

    s = s01[:_N] + s01[_N:]
    r = 1.0 / (s + 1e-16)
    r2d = r[:, None]

    x2 = pl.pallas_call(
        _mid_kernel,
        out_shape=jax.ShapeDtypeStruct((_N, _HID), _f32),
    )(y1un, r2d, W2)

    spmm_k = pl.kernel(
        _spmm_kernel,
        out_type=[
            jax.ShapeDtypeStruct((_NC, _N, _HID), _f32),
            jax.ShapeDtypeStruct((_E,), _f32),
        ],
        mesh=plsc.VectorSubcoreMesh(core_axis_name="c", subcore_axis_name="s"),
        compiler_params=pltpu.CompilerParams(needs_layout_passes=False),
        scratch_types=(
            [
                pltpu.VMEM((_CH,), _i32),
                pltpu.VMEM((_CH,), _i32),
                pltpu.VMEM((_CH, _HID), _f32),
                pltpu.VMEM((_CH,), _f32),
                pltpu.VMEM((_CH,), _f32),
                pltpu.VMEM((_N,), _f32),
                pltpu.VMEM_SHARED((_N, _HID), _f32),
                pltpu.SemaphoreType.DMA,
                pltpu.SemaphoreType.DMA,
            ]
        ),
    )
    y2un, adj = spmm_k(x2, src, dst, ex, r, zeros)

    y2 = pl.pallas_call(
        _fin_kernel,
        out_shape=jax.ShapeDtypeStruct((_N, _HID), _f32),
    )(y2un, r2d)

    return (y2, h, adj)


# speedup vs baseline: 7.1349x; 1.2828x over previous
import jax
import jax.numpy as jnp
from jax import lax
from jax.experimental import pallas as pl
from jax.experimental.pallas import tpu as pltpu
from jax.experimental.pallas import tpu_sc as plsc

_N = 10000
_E = 320000
_HGL = 64
_HID = 128
_NC = 2
_NS = 16
_EPT = _E // (_NC * _NS)
_CH = 80
_NCHUNK = _EPT // _CH
_RPT = 624
_TAIL = _N - _NS * _RPT

_f32 = jnp.float32
_i32 = jnp.int32


def _mm_kernel(x_ref, wgl_ref, w1_ref, h_ref, x1_ref):
    x = x_ref[...]
    h_ref[...] = x @ wgl_ref[...]
    x1_ref[...] = x @ w1_ref[...]


def _mid_kernel(yun_ref, r_ref, w2_ref, x2_ref):
    y = yun_ref[0] + yun_ref[1]
    y = jnp.maximum(y * r_ref[...], 0.0)
    x2_ref[...] = y @ w2_ref[...]


def _fin_kernel(yun_ref, r_ref, out_ref):
    out_ref[...] = (yun_ref[0] + yun_ref[1]) * r_ref[...]


def _scale_rows(xs, exc):
    def gbody(g, carry):
        ev = exc[pl.ds(g * 16, 16)]
        for j in range(16):
            e = g * 16 + j
            w = ev[j]
            for k in range(_HID // 16):
                xs[e, pl.ds(k * 16, 16)] = xs[e, pl.ds(k * 16, 16)] * w
        return carry

    lax.fori_loop(0, _CH // 16, gbody, 0)


def _acc_init(z_hbm, z1_hbm, y_acc, s_acc, sb, s):
    pltpu.sync_copy(z_hbm.at[pl.ds(s * _RPT, _RPT), :],
                    y_acc.at[pl.ds(s * _RPT, _RPT), :])
    pltpu.sync_copy(z1_hbm.at[pl.ds(s * _RPT, _RPT)], sb)
    pltpu.sync_copy(sb, s_acc.at[pl.ds(s * _RPT, _RPT)])

    @pl.when(s == _NS - 1)
    def _():
        pltpu.sync_copy(z_hbm.at[pl.ds(_NS * _RPT, _TAIL), :],
                        y_acc.at[pl.ds(_NS * _RPT, _TAIL), :])
        pltpu.sync_copy(sb.at[pl.ds(0, _TAIL)],
                        s_acc.at[pl.ds(_NS * _RPT, _TAIL)])

    plsc.subcore_barrier()


def _edge_kernel(hp_hbm, x_hbm, src_hbm, dst_hbm, a_hbm, z_hbm, z1_hbm,
                 yun_hbm, s01_hbm, ex_hbm,
                 sidxA, didxA, s2A, d2A, xsA,
                 sidxB, didxB, s2B, d2B, xsB,
                 hs, hd, exc, a_buf, sb, y_acc, s_acc,
                 semXA, semSA, semDA, semXB, semSB, semDB, semw):
    c = lax.axis_index("c")
    s = lax.axis_index("s")
    base = (c * _NS + s) * _EPT

    pltpu.sync_copy(a_hbm, a_buf)
    _acc_init(z_hbm, z1_hbm, y_acc, s_acc, sb, s)

    iota = lax.iota(_i32, 16)
    A = (sidxA, didxA, s2A, d2A, xsA, semXA, semSA, semDA)
    B = (sidxB, didxB, s2B, d2B, xsB, semXB, semSB, semDB)

    def load_idx(j, buf):
        sidx, didx, s2, d2 = buf[0], buf[1], buf[2], buf[3]
        off = base + j * _CH
        pltpu.sync_copy(src_hbm.at[pl.ds(off, _CH)], sidx)
        pltpu.sync_copy(dst_hbm.at[pl.ds(off, _CH)], didx)

        def gb(g, carry):
            s16 = sidx[pl.ds(g * 16, 16)]
            d16 = didx[pl.ds(g * 16, 16)]
            s2[pl.ds(g * 16, 16)] = jnp.right_shift(s16, 1)
            d2[pl.ds(g * 16, 16)] = jnp.right_shift(d16, 1)
            return carry

        lax.fori_loop(0, _CH // 16, gb, 0)

    def issue_x(buf):
        pltpu.async_copy(x_hbm.at[buf[0]], buf[4], buf[5])

    def wait_x(buf):
        pltpu.make_async_copy(x_hbm.at[buf[0]], buf[4], buf[5]).wait()

    def issue_h(buf):
        pltpu.async_copy(hp_hbm.at[buf[2]], hs, buf[6])
        pltpu.async_copy(hp_hbm.at[buf[3]], hd, buf[7])

    def wait_h(buf):
        pltpu.make_async_copy(hp_hbm.at[buf[2]], hs, buf[6]).wait()
        pltpu.make_async_copy(hp_hbm.at[buf[3]], hd, buf[7]).wait()

    def score(buf):
        sidx, didx = buf[0], buf[1]

        def gbody(g, carry):
            s16 = sidx[pl.ds(g * 16, 16)]
            d16 = didx[pl.ds(g * 16, 16)]
            ps = (s16 & 1) * _HGL
            pd = (d16 & 1) * _HGL
            rows = g * 16 + iota

            def fbody(fb, acc):
                av = a_buf[pl.ds(fb * 16, 16)]
                for j in range(16):
                    cs = ps + (fb * 16 + j)
                    cd = pd + (fb * 16 + j)
                    hsv = plsc.load_gather(hs, [rows, cs])
                    hdv = plsc.load_gather(hd, [rows, cd])
                    acc = acc + jnp.abs(hsv - hdv) * av[j]
                return acc

            acc = lax.fori_loop(0, _HGL // 16, fbody, jnp.zeros((16,), _f32))
            exc[pl.ds(g * 16, 16)] = jnp.exp(jnp.maximum(acc, 0.0))
            return carry

        lax.fori_loop(0, _CH // 16, gbody, 0)

    def scatter(j, buf):
        didx, xs = buf[1], buf[4]
        off = base + j * _CH
        w = pltpu.async_copy(exc, ex_hbm.at[pl.ds(off, _CH)], semw)
        w.wait()
        w = pltpu.async_copy(xs, y_acc.at[didx], semw, add=True)
        w.wait()
        w = pltpu.async_copy(exc, s_acc.at[didx], semw, add=True)
        w.wait()

    def half(j, cur, nxt, last):
        if not last:
            load_idx(j + 1, nxt)
            issue_x(nxt)
        wait_x(cur)
        wait_h(cur)
        score(cur)
        if not last:
            issue_h(nxt)
        _scale_rows(cur[4], exc)
        scatter(j, cur)

    load_idx(0, A)
    issue_x(A)
    issue_h(A)

    def pair(k, carry):
        half(2 * k, A, B, False)
        half(2 * k + 1, B, A, False)
        return carry

    lax.fori_loop(0, (_NCHUNK - 1) // 2, pair, 0)
    half(_NCHUNK - 1, A, B, True)

    plsc.subcore_barrier()

    pltpu.sync_copy(y_acc.at[pl.ds(s * _RPT, _RPT), :],
                    yun_hbm.at[c, pl.ds(s * _RPT, _RPT), :])
    pltpu.sync_copy(s_acc.at[pl.ds(s * _RPT, _RPT)], sb)
    pltpu.sync_copy(sb, s01_hbm.at[pl.ds(c * _N + s * _RPT, _RPT)])

    @pl.when(s == _NS - 1)
    def _():
        pltpu.sync_copy(y_acc.at[pl.ds(_NS * _RPT, _TAIL), :],
                        yun_hbm.at[c, pl.ds(_NS * _RPT, _TAIL), :])
        pltpu.sync_copy(s_acc.at[pl.ds(_NS * _RPT, _TAIL)],
                        sb.at[pl.ds(0, _TAIL)])
        pltpu.sync_copy(sb.at[pl.ds(0, _TAIL)],
                        s01_hbm.at[pl.ds(c * _N + _NS * _RPT, _TAIL)])


def _spmm_kernel(x_hbm, src_hbm, dst_hbm, exg_hbm, r_hbm, z_hbm,
                 yun_hbm, adj_hbm,
                 sidx, didx, xs, exc,
                 adjc, r_vmem, y_acc, sem, semw):
    c = lax.axis_index("c")
    s = lax.axis_index("s")
    base = (c * _NS + s) * _EPT

    pltpu.sync_copy(r_hbm, r_vmem)
    pltpu.sync_copy(z_hbm.at[pl.ds(s * _RPT, _RPT), :],
                    y_acc.at[pl.ds(s * _RPT, _RPT), :])

    @pl.when(s == _NS - 1)
    def _():
        pltpu.sync_copy(z_hbm.at[pl.ds(_NS * _RPT, _TAIL), :],
                        y_acc.at[pl.ds(_NS * _RPT, _TAIL), :])

    plsc.subcore_barrier()

    def step(j, carry):
        off = base + j * _CH
        pltpu.sync_copy(src_hbm.at[pl.ds(off, _CH)], sidx)
        pltpu.sync_copy(dst_hbm.at[pl.ds(off, _CH)], didx)
        w = pltpu.async_copy(exg_hbm.at[pl.ds(off, _CH)], exc, sem)
        w.wait()
        w = pltpu.async_copy(x_hbm.at[sidx], xs, sem)
        w.wait()

        def gbody(g, carry2):
            d16 = didx[pl.ds(g * 16, 16)]
            rv = plsc.load_gather(r_vmem, [d16])
            adjc[pl.ds(g * 16, 16)] = exc[pl.ds(g * 16, 16)] * rv
            return carry2

        lax.fori_loop(0, _CH // 16, gbody, 0)
        _scale_rows(xs, exc)
        w = pltpu.async_copy(adjc, adj_hbm.at[pl.ds(off, _CH)], semw)
        w.wait()
        w = pltpu.async_copy(xs, y_acc.at[didx], semw, add=True)
        w.wait()
        return carry

    lax.fori_loop(0, _NCHUNK, step, 0)

    plsc.subcore_barrier()

    pltpu.sync_copy(y_acc.at[pl.ds(s * _RPT, _RPT), :],
                    yun_hbm.at[c, pl.ds(s * _RPT, _RPT), :])

    @pl.when(s == _NS - 1)
    def _():
        pltpu.sync_copy(y_acc.at[pl.ds(_NS * _RPT, _TAIL), :],
                        yun_hbm.at[c, pl.ds(_NS * _RPT, _TAIL), :])


def kernel(inputs, edge, W_gl, a_gl, W1, W2):
    src = edge[0].astype(_i32)
    dst = edge[1].astype(_i32)
    zeros = jnp.zeros((_N, _HID), _f32)
    zeros1 = jnp.zeros((_N,), _f32)

    h, x1 = pl.pallas_call(
        _mm_kernel,
        out_shape=[
            jax.ShapeDtypeStruct((_N, _HGL), _f32),
            jax.ShapeDtypeStruct((_N, _HID), _f32),
        ],
    )(inputs, W_gl, W1)
    hp = h.reshape(_N // 2, 2 * _HGL)

    edge_k = pl.kernel(
        _edge_kernel,
        out_type=[
            jax.ShapeDtypeStruct((_NC, _N, _HID), _f32),
            jax.ShapeDtypeStruct((_NC * _N,), _f32),
            jax.ShapeDtypeStruct((_E,), _f32),
        ],
        mesh=plsc.VectorSubcoreMesh(core_axis_name="c", subcore_axis_name="s"),
        compiler_params=pltpu.CompilerParams(needs_layout_passes=False),
        scratch_types=(
            2 * [
                pltpu.VMEM((_CH,), _i32),
                pltpu.VMEM((_CH,), _i32),
                pltpu.VMEM((_CH,), _i32),
                pltpu.VMEM((_CH,), _i32),
                pltpu.VMEM((_CH, _HID), _f32),
            ]
            + [
                pltpu.VMEM((_CH, 2 * _HGL), _f32),
                pltpu.VMEM((_CH, 2 * _HGL), _f32),
                pltpu.VMEM((_CH,), _f32),
                pltpu.VMEM((_HGL,), _f32),
                pltpu.VMEM((_RPT,), _f32),
                pltpu.VMEM_SHARED((_N, _HID), _f32),
                pltpu.VMEM_SHARED((_N,), _f32),
            ]
            + 6 * [pltpu.SemaphoreType.DMA]
            + [pltpu.SemaphoreType.DMA]
        ),
    )
    y1un, s01, ex = edge_k(hp, x1, src, dst, a_gl, zeros, zeros1)

    s = s01[:_N] + s01[_N:]
    r = 1.0 / (s + 1e-16)
    r2d = r[:, None]

    x2 = pl.pallas_call(
        _mid_kernel,
        out_shape=jax.ShapeDtypeStruct((_N, _HID), _f32),
    )(y1un, r2d, W2)

    spmm_k = pl.kernel(
        _spmm_kernel,
        out_type=[
            jax.ShapeDtypeStruct((_NC, _N, _HID), _f32),
            jax.ShapeDtypeStruct((_E,), _f32),
        ],
        mesh=plsc.VectorSubcoreMesh(core_axis_name="c", subcore_axis_name="s"),
        compiler_params=pltpu.CompilerParams(needs_layout_passes=False),
        scratch_types=(
            [
                pltpu.VMEM((_CH,), _i32),
                pltpu.VMEM((_CH,), _i32),
                pltpu.VMEM((_CH, _HID), _f32),
                pltpu.VMEM((_CH,), _f32),
                pltpu.VMEM((_CH,), _f32),
                pltpu.VMEM((_N,), _f32),
                pltpu.VMEM_SHARED((_N, _HID), _f32),
                pltpu.SemaphoreType.DMA,
                pltpu.SemaphoreType.DMA,
            ]
        ),
    )
    y2un, adj = spmm_k(x2, src, dst, ex, r, zeros)

    y2 = pl.pallas_call(
        _fin_kernel,
        out_shape=jax.ShapeDtypeStruct((_N, _HID), _f32),
    )(y2un, r2d)

    return (y2, h, adj)


# speedup vs baseline: 8.2179x; 1.1518x over previous
import jax
import jax.numpy as jnp
from jax import lax
from jax.experimental import pallas as pl
from jax.experimental.pallas import tpu as pltpu
from jax.experimental.pallas import tpu_sc as plsc

_N = 10000
_E = 320000
_HGL = 64
_HID = 128
_NC = 2
_NS = 16
_EPT = _E // (_NC * _NS)
_CH = 80
_NCHUNK = _EPT // _CH
_RPT = 624
_TAIL = _N - _NS * _RPT

_f32 = jnp.float32
_i32 = jnp.int32


def _mm_kernel(x_ref, wgl_ref, w1_ref, h_ref, x1_ref):
    x = x_ref[...]
    h_ref[...] = x @ wgl_ref[...]
    x1_ref[...] = x @ w1_ref[...]


def _mid_kernel(yun_ref, r_ref, w2_ref, x2_ref):
    y = yun_ref[0] + yun_ref[1]
    y = jnp.maximum(y * r_ref[...], 0.0)
    x2_ref[...] = y @ w2_ref[...]


def _fin_kernel(yun_ref, r_ref, out_ref):
    out_ref[...] = (yun_ref[0] + yun_ref[1]) * r_ref[...]


def _scale_rows(xs, exc):
    def gbody(g, carry):
        ev = exc[pl.ds(g * 16, 16)]
        for j in range(16):
            e = g * 16 + j
            w = ev[j]
            for k in range(_HID // 16):
                xs[e, pl.ds(k * 16, 16)] = xs[e, pl.ds(k * 16, 16)] * w
        return carry

    lax.fori_loop(0, _CH // 16, gbody, 0)


def _acc_init(z_hbm, z1_hbm, y_acc, s_acc, sb, s):
    pltpu.sync_copy(z_hbm.at[pl.ds(s * _RPT, _RPT), :],
                    y_acc.at[pl.ds(s * _RPT, _RPT), :])
    pltpu.sync_copy(z1_hbm.at[pl.ds(s * _RPT, _RPT)], sb)
    pltpu.sync_copy(sb, s_acc.at[pl.ds(s * _RPT, _RPT)])

    @pl.when(s == _NS - 1)
    def _():
        pltpu.sync_copy(z_hbm.at[pl.ds(_NS * _RPT, _TAIL), :],
                        y_acc.at[pl.ds(_NS * _RPT, _TAIL), :])
        pltpu.sync_copy(sb.at[pl.ds(0, _TAIL)],
                        s_acc.at[pl.ds(_NS * _RPT, _TAIL)])

    plsc.subcore_barrier()


def _edge_kernel(hp_hbm, x_hbm, src_hbm, dst_hbm, a_hbm, z_hbm, z1_hbm,
                 yun_hbm, s01_hbm, ex_hbm,
                 sidxA, didxA, s2A, d2A, xsA,
                 sidxB, didxB, s2B, d2B, xsB,
                 hs, hd, exc, a_buf, sb, y_acc, s_acc,
                 semXA, semSA, semDA, semXB, semSB, semDB, semw):
    c = lax.axis_index("c")
    s = lax.axis_index("s")
    base = (c * _NS + s) * _EPT

    pltpu.sync_copy(a_hbm, a_buf)
    _acc_init(z_hbm, z1_hbm, y_acc, s_acc, sb, s)

    iota = lax.iota(_i32, 16)
    A = (sidxA, didxA, s2A, d2A, xsA, semXA, semSA, semDA)
    B = (sidxB, didxB, s2B, d2B, xsB, semXB, semSB, semDB)

    def load_idx(j, buf):
        sidx, didx, s2, d2 = buf[0], buf[1], buf[2], buf[3]
        off = base + j * _CH
        pltpu.sync_copy(src_hbm.at[pl.ds(off, _CH)], sidx)
        pltpu.sync_copy(dst_hbm.at[pl.ds(off, _CH)], didx)

        def gb(g, carry):
            s16 = sidx[pl.ds(g * 16, 16)]
            d16 = didx[pl.ds(g * 16, 16)]
            s2[pl.ds(g * 16, 16)] = jnp.right_shift(s16, 1)
            d2[pl.ds(g * 16, 16)] = jnp.right_shift(d16, 1)
            return carry

        lax.fori_loop(0, _CH // 16, gb, 0)

    def issue_x(buf):
        pltpu.async_copy(x_hbm.at[buf[0]], buf[4], buf[5])

    def wait_x(buf):
        pltpu.make_async_copy(x_hbm.at[buf[0]], buf[4], buf[5]).wait()

    def issue_h(buf):
        pltpu.async_copy(hp_hbm.at[buf[2]], hs, buf[6])
        pltpu.async_copy(hp_hbm.at[buf[3]], hd, buf[7])

    def wait_h(buf):
        pltpu.make_async_copy(hp_hbm.at[buf[2]], hs, buf[6]).wait()
        pltpu.make_async_copy(hp_hbm.at[buf[3]], hd, buf[7]).wait()

    def score(buf):
        sidx, didx = buf[0], buf[1]

        def gbody(g, carry):
            s16 = sidx[pl.ds(g * 16, 16)]
            d16 = didx[pl.ds(g * 16, 16)]
            ps = (s16 & 1) * _HGL
            pd = (d16 & 1) * _HGL
            rows = g * 16 + iota

            def fbody(fb, acc):
                av = a_buf[pl.ds(fb * 16, 16)]
                for j in range(16):
                    cs = ps + (fb * 16 + j)
                    cd = pd + (fb * 16 + j)
                    hsv = plsc.load_gather(hs, [rows, cs])
                    hdv = plsc.load_gather(hd, [rows, cd])
                    acc = acc + jnp.abs(hsv - hdv) * av[j]
                return acc

            acc = lax.fori_loop(0, _HGL // 16, fbody, jnp.zeros((16,), _f32))
            exc[pl.ds(g * 16, 16)] = jnp.exp(jnp.maximum(acc, 0.0))
            return carry

        lax.fori_loop(0, _CH // 16, gbody, 0)

    def scatter(j, buf):
        didx, xs = buf[1], buf[4]
        off = base + j * _CH
        w = pltpu.async_copy(exc, ex_hbm.at[pl.ds(off, _CH)], semw)
        w.wait()
        w = pltpu.async_copy(xs, y_acc.at[didx], semw, add=True)
        w.wait()
        w = pltpu.async_copy(exc, s_acc.at[didx], semw, add=True)
        w.wait()

    def half(j, cur, nxt, last):
        if not last:
            load_idx(j + 1, nxt)
            issue_x(nxt)
        wait_x(cur)
        wait_h(cur)
        score(cur)
        if not last:
            issue_h(nxt)
        _scale_rows(cur[4], exc)
        scatter(j, cur)

    load_idx(0, A)
    issue_x(A)
    issue_h(A)

    def pair(k, carry):
        half(2 * k, A, B, False)
        half(2 * k + 1, B, A, False)
        return carry

    lax.fori_loop(0, (_NCHUNK - 1) // 2, pair, 0)
    half(_NCHUNK - 1, A, B, True)

    plsc.subcore_barrier()

    pltpu.sync_copy(y_acc.at[pl.ds(s * _RPT, _RPT), :],
                    yun_hbm.at[c, pl.ds(s * _RPT, _RPT), :])
    pltpu.sync_copy(s_acc.at[pl.ds(s * _RPT, _RPT)], sb)
    pltpu.sync_copy(sb, s01_hbm.at[pl.ds(c * _N + s * _RPT, _RPT)])

    @pl.when(s == _NS - 1)
    def _():
        pltpu.sync_copy(y_acc.at[pl.ds(_NS * _RPT, _TAIL), :],
                        yun_hbm.at[c, pl.ds(_NS * _RPT, _TAIL), :])
        pltpu.sync_copy(s_acc.at[pl.ds(_NS * _RPT, _TAIL)],
                        sb.at[pl.ds(0, _TAIL)])
        pltpu.sync_copy(sb.at[pl.ds(0, _TAIL)],
                        s01_hbm.at[pl.ds(c * _N + _NS * _RPT, _TAIL)])


def _spmm_kernel(x_hbm, src_hbm, dst_hbm, exg_hbm, r_hbm, z_hbm,
                 yun_hbm, adj_hbm,
                 sidxA, didxA, xsA, excA,
                 sidxB, didxB, xsB, excB,
                 adjc, r_vmem, y_acc,
                 semXA, semEA, semXB, semEB, semw):
    c = lax.axis_index("c")
    s = lax.axis_index("s")
    base = (c * _NS + s) * _EPT

    pltpu.sync_copy(r_hbm, r_vmem)
    pltpu.sync_copy(z_hbm.at[pl.ds(s * _RPT, _RPT), :],
                    y_acc.at[pl.ds(s * _RPT, _RPT), :])

    @pl.when(s == _NS - 1)
    def _():
        pltpu.sync_copy(z_hbm.at[pl.ds(_NS * _RPT, _TAIL), :],
                        y_acc.at[pl.ds(_NS * _RPT, _TAIL), :])

    plsc.subcore_barrier()

    A = (sidxA, didxA, xsA, excA, semXA, semEA)
    B = (sidxB, didxB, xsB, excB, semXB, semEB)

    def load_idx(j, buf):
        off = base + j * _CH
        pltpu.sync_copy(src_hbm.at[pl.ds(off, _CH)], buf[0])
        pltpu.sync_copy(dst_hbm.at[pl.ds(off, _CH)], buf[1])
        pltpu.async_copy(exg_hbm.at[pl.ds(off, _CH)], buf[3], buf[5])
        pltpu.async_copy(x_hbm.at[buf[0]], buf[2], buf[4])

    def wait_loads(j, buf):
        off = base + j * _CH
        pltpu.make_async_copy(exg_hbm.at[pl.ds(off, _CH)], buf[3],
                              buf[5]).wait()
        pltpu.make_async_copy(x_hbm.at[buf[0]], buf[2], buf[4]).wait()

    def half(j, cur, nxt, last):
        if not last:
            load_idx(j + 1, nxt)
        wait_loads(j, cur)
        sidx, didx, xs, exc = cur[0], cur[1], cur[2], cur[3]

        def gbody(g, carry2):
            d16 = didx[pl.ds(g * 16, 16)]
            rv = plsc.load_gather(r_vmem, [d16])
            adjc[pl.ds(g * 16, 16)] = exc[pl.ds(g * 16, 16)] * rv
            return carry2

        lax.fori_loop(0, _CH // 16, gbody, 0)
        _scale_rows(xs, exc)
        off = base + j * _CH
        w = pltpu.async_copy(adjc, adj_hbm.at[pl.ds(off, _CH)], semw)
        w.wait()
        w = pltpu.async_copy(xs, y_acc.at[didx], semw, add=True)
        w.wait()

    load_idx(0, A)

    def pair(k, carry):
        half(2 * k, A, B, False)
        half(2 * k + 1, B, A, False)
        return carry

    lax.fori_loop(0, (_NCHUNK - 1) // 2, pair, 0)
    half(_NCHUNK - 1, A, B, True)

    plsc.subcore_barrier()

    pltpu.sync_copy(y_acc.at[pl.ds(s * _RPT, _RPT), :],
                    yun_hbm.at[c, pl.ds(s * _RPT, _RPT), :])

    @pl.when(s == _NS - 1)
    def _():
        pltpu.sync_copy(y_acc.at[pl.ds(_NS * _RPT, _TAIL), :],
                        yun_hbm.at[c, pl.ds(_NS * _RPT, _TAIL), :])


def kernel(inputs, edge, W_gl, a_gl, W1, W2):
    src = edge[0].astype(_i32)
    dst = edge[1].astype(_i32)
    zeros = jnp.zeros((_N, _HID), _f32)
    zeros1 = jnp.zeros((_N,), _f32)

    h, x1 = pl.pallas_call(
        _mm_kernel,
        out_shape=[
            jax.ShapeDtypeStruct((_N, _HGL), _f32),
            jax.ShapeDtypeStruct((_N, _HID), _f32),
        ],
    )(inputs, W_gl, W1)
    hp = h.reshape(_N // 2, 2 * _HGL)

    edge_k = pl.kernel(
        _edge_kernel,
        out_type=[
            jax.ShapeDtypeStruct((_NC, _N, _HID), _f32),
            jax.ShapeDtypeStruct((_NC * _N,), _f32),
            jax.ShapeDtypeStruct((_E,), _f32),
        ],
        mesh=plsc.VectorSubcoreMesh(core_axis_name="c", subcore_axis_name="s"),
        compiler_params=pltpu.CompilerParams(needs_layout_passes=False),
        scratch_types=(
            2 * [
                pltpu.VMEM((_CH,), _i32),
                pltpu.VMEM((_CH,), _i32),
                pltpu.VMEM((_CH,), _i32),
                pltpu.VMEM((_CH,), _i32),
                pltpu.VMEM((_CH, _HID), _f32),
            ]
            + [
                pltpu.VMEM((_CH, 2 * _HGL), _f32),
                pltpu.VMEM((_CH, 2 * _HGL), _f32),
                pltpu.VMEM((_CH,), _f32),
                pltpu.VMEM((_HGL,), _f32),
                pltpu.VMEM((_RPT,), _f32),
                pltpu.VMEM_SHARED((_N, _HID), _f32),
                pltpu.VMEM_SHARED((_N,), _f32),
            ]
            + 6 * [pltpu.SemaphoreType.DMA]
            + [pltpu.SemaphoreType.DMA]
        ),
    )
    y1un, s01, ex = edge_k(hp, x1, src, dst, a_gl, zeros, zeros1)

    s = s01[:_N] + s01[_N:]
    r = 1.0 / (s + 1e-16)
    r2d = r[:, None]

    x2 = pl.pallas_call(
        _mid_kernel,
        out_shape=jax.ShapeDtypeStruct((_N, _HID), _f32),
    )(y1un, r2d, W2)

    spmm_k = pl.kernel(
        _spmm_kernel,
        out_type=[
            jax.ShapeDtypeStruct((_NC, _N, _HID), _f32),
            jax.ShapeDtypeStruct((_E,), _f32),
        ],
        mesh=plsc.VectorSubcoreMesh(core_axis_name="c", subcore_axis_name="s"),
        compiler_params=pltpu.CompilerParams(needs_layout_passes=False),
        scratch_types=(
            2 * [
                pltpu.VMEM((_CH,), _i32),
                pltpu.VMEM((_CH,), _i32),
                pltpu.VMEM((_CH, _HID), _f32),
                pltpu.VMEM((_CH,), _f32),
            ]
            + [
                pltpu.VMEM((_CH,), _f32),
                pltpu.VMEM((_N,), _f32),
                pltpu.VMEM_SHARED((_N, _HID), _f32),
            ]
            + 4 * [pltpu.SemaphoreType.DMA]
            + [pltpu.SemaphoreType.DMA]
        ),
    )
    y2un, adj = spmm_k(x2, src, dst, ex, r, zeros)

    y2 = pl.pallas_call(
        _fin_kernel,
        out_shape=jax.ShapeDtypeStruct((_N, _HID), _f32),
    )(y2un, r2d)

    return (y2, h, adj)


# speedup vs baseline: 8.5244x; 1.0373x over previous
import jax
import jax.numpy as jnp
from jax import lax
from jax.experimental import pallas as pl
from jax.experimental.pallas import tpu as pltpu
from jax.experimental.pallas import tpu_sc as plsc

_N = 10000
_E = 320000
_HGL = 64
_HID = 128
_NC = 2
_NS = 16
_EPT = _E // (_NC * _NS)
_CH = 80
_NCHUNK = _EPT // _CH
_RPT = 624
_TAIL = _N - _NS * _RPT

_f32 = jnp.float32
_i32 = jnp.int32


def _mm_kernel(x_ref, wgl_ref, w1_ref, h_ref, x1_ref):
    x = x_ref[...]
    h_ref[...] = x @ wgl_ref[...]
    x1_ref[...] = x @ w1_ref[...]


def _mid_kernel(yun_ref, r_ref, w2_ref, x2_ref):
    y = yun_ref[0] + yun_ref[1]
    y = jnp.maximum(y * r_ref[...], 0.0)
    x2_ref[...] = y @ w2_ref[...]


def _fin_kernel(yun_ref, r_ref, out_ref):
    out_ref[...] = (yun_ref[0] + yun_ref[1]) * r_ref[...]


def _scale_rows(xs, exc):
    def gbody(g, carry):
        ev = exc[pl.ds(g * 16, 16)]
        for j in range(16):
            e = g * 16 + j
            w = ev[j]
            for k in range(_HID // 16):
                xs[e, pl.ds(k * 16, 16)] = xs[e, pl.ds(k * 16, 16)] * w
        return carry

    lax.fori_loop(0, _CH // 16, gbody, 0)


def _acc_init(z_hbm, z1_hbm, y_acc, s_acc, sb, s):
    pltpu.sync_copy(z_hbm.at[pl.ds(s * _RPT, _RPT), :],
                    y_acc.at[pl.ds(s * _RPT, _RPT), :])
    pltpu.sync_copy(z1_hbm.at[pl.ds(s * _RPT, _RPT)], sb)
    pltpu.sync_copy(sb, s_acc.at[pl.ds(s * _RPT, _RPT)])

    @pl.when(s == _NS - 1)
    def _():
        pltpu.sync_copy(z_hbm.at[pl.ds(_NS * _RPT, _TAIL), :],
                        y_acc.at[pl.ds(_NS * _RPT, _TAIL), :])
        pltpu.sync_copy(sb.at[pl.ds(0, _TAIL)],
                        s_acc.at[pl.ds(_NS * _RPT, _TAIL)])

    plsc.subcore_barrier()


def _edge_kernel(hp_hbm, x_hbm, src_hbm, dst_hbm, a_hbm, z_hbm, z1_hbm,
                 yun_hbm, s01_hbm, ex_hbm,
                 sidxA, didxA, s2A, d2A, xsA,
                 sidxB, didxB, s2B, d2B, xsB,
                 hs, hd, exc, a_buf, sb, y_acc, s_acc,
                 semXA, semSA, semDA, semXB, semSB, semDB,
                 semIsA, semIdA, semIsB, semIdB, semw):
    c = lax.axis_index("c")
    s = lax.axis_index("s")
    base = (c * _NS + s) * _EPT

    pltpu.sync_copy(a_hbm, a_buf)
    _acc_init(z_hbm, z1_hbm, y_acc, s_acc, sb, s)

    iota = lax.iota(_i32, 16)
    A = (sidxA, didxA, s2A, d2A, xsA, semXA, semSA, semDA, semIsA, semIdA)
    B = (sidxB, didxB, s2B, d2B, xsB, semXB, semSB, semDB, semIsB, semIdB)

    def issue_idx(j, buf):
        off = base + j * _CH
        pltpu.async_copy(src_hbm.at[pl.ds(off, _CH)], buf[0], buf[8])
        pltpu.async_copy(dst_hbm.at[pl.ds(off, _CH)], buf[1], buf[9])

    def wait_idx(j, buf):
        off = base + j * _CH
        pltpu.make_async_copy(src_hbm.at[pl.ds(off, _CH)], buf[0],
                              buf[8]).wait()
        pltpu.make_async_copy(dst_hbm.at[pl.ds(off, _CH)], buf[1],
                              buf[9]).wait()

    def shift_idx(buf):
        sidx, didx, s2, d2 = buf[0], buf[1], buf[2], buf[3]

        def gb(g, carry):
            s16 = sidx[pl.ds(g * 16, 16)]
            d16 = didx[pl.ds(g * 16, 16)]
            s2[pl.ds(g * 16, 16)] = jnp.right_shift(s16, 1)
            d2[pl.ds(g * 16, 16)] = jnp.right_shift(d16, 1)
            return carry

        lax.fori_loop(0, _CH // 16, gb, 0)

    def issue_x(buf):
        pltpu.async_copy(x_hbm.at[buf[0]], buf[4], buf[5])

    def wait_x(buf):
        pltpu.make_async_copy(x_hbm.at[buf[0]], buf[4], buf[5]).wait()

    def issue_h(buf):
        pltpu.async_copy(hp_hbm.at[buf[2]], hs, buf[6])
        pltpu.async_copy(hp_hbm.at[buf[3]], hd, buf[7])

    def wait_h(buf):
        pltpu.make_async_copy(hp_hbm.at[buf[2]], hs, buf[6]).wait()
        pltpu.make_async_copy(hp_hbm.at[buf[3]], hd, buf[7]).wait()

    def score(buf):
        sidx, didx = buf[0], buf[1]

        def gbody(g, carry):
            s16 = sidx[pl.ds(g * 16, 16)]
            d16 = didx[pl.ds(g * 16, 16)]
            ps = (s16 & 1) * _HGL
            pd = (d16 & 1) * _HGL
            rows = g * 16 + iota

            def fbody(fb, acc):
                av = a_buf[pl.ds(fb * 16, 16)]
                for j in range(16):
                    cs = ps + (fb * 16 + j)
                    cd = pd + (fb * 16 + j)
                    hsv = plsc.load_gather(hs, [rows, cs])
                    hdv = plsc.load_gather(hd, [rows, cd])
                    acc = acc + jnp.abs(hsv - hdv) * av[j]
                return acc

            acc = lax.fori_loop(0, _HGL // 16, fbody, jnp.zeros((16,), _f32))
            exc[pl.ds(g * 16, 16)] = jnp.exp(jnp.maximum(acc, 0.0))
            return carry

        lax.fori_loop(0, _CH // 16, gbody, 0)

    def scatter(j, buf):
        didx, xs = buf[1], buf[4]
        off = base + j * _CH
        w = pltpu.async_copy(exc, ex_hbm.at[pl.ds(off, _CH)], semw)
        w.wait()
        w = pltpu.async_copy(xs, y_acc.at[didx], semw, add=True)
        w.wait()
        w = pltpu.async_copy(exc, s_acc.at[didx], semw, add=True)
        w.wait()

    def half(j, cur, nxt, last):
        if not last:
            wait_idx(j + 1, nxt)
            shift_idx(nxt)
            issue_x(nxt)
        wait_x(cur)
        wait_h(cur)
        score(cur)
        if not last:
            issue_h(nxt)
        _scale_rows(cur[4], exc)
        scatter(j, cur)
        if not last:
            @pl.when(j + 2 < _NCHUNK)
            def _():
                issue_idx(j + 2, cur)

    pltpu.sync_copy(src_hbm.at[pl.ds(base, _CH)], sidxA)
    pltpu.sync_copy(dst_hbm.at[pl.ds(base, _CH)], didxA)
    shift_idx(A)
    issue_x(A)
    issue_h(A)
    issue_idx(1, B)

    def pair(k, carry):
        half(2 * k, A, B, False)
        half(2 * k + 1, B, A, False)
        return carry

    lax.fori_loop(0, (_NCHUNK - 1) // 2, pair, 0)
    half(_NCHUNK - 1, A, B, True)

    plsc.subcore_barrier()

    pltpu.sync_copy(y_acc.at[pl.ds(s * _RPT, _RPT), :],
                    yun_hbm.at[c, pl.ds(s * _RPT, _RPT), :])
    pltpu.sync_copy(s_acc.at[pl.ds(s * _RPT, _RPT)], sb)
    pltpu.sync_copy(sb, s01_hbm.at[pl.ds(c * _N + s * _RPT, _RPT)])

    @pl.when(s == _NS - 1)
    def _():
        pltpu.sync_copy(y_acc.at[pl.ds(_NS * _RPT, _TAIL), :],
                        yun_hbm.at[c, pl.ds(_NS * _RPT, _TAIL), :])
        pltpu.sync_copy(s_acc.at[pl.ds(_NS * _RPT, _TAIL)],
                        sb.at[pl.ds(0, _TAIL)])
        pltpu.sync_copy(sb.at[pl.ds(0, _TAIL)],
                        s01_hbm.at[pl.ds(c * _N + _NS * _RPT, _TAIL)])


def _spmm_kernel(x_hbm, src_hbm, dst_hbm, exg_hbm, r_hbm, z_hbm,
                 yun_hbm, adj_hbm,
                 sidxA, didxA, xsA, excA,
                 sidxB, didxB, xsB, excB,
                 adjc, r_vmem, y_acc,
                 semXA, semEA, semXB, semEB, semw):
    c = lax.axis_index("c")
    s = lax.axis_index("s")
    base = (c * _NS + s) * _EPT

    pltpu.sync_copy(r_hbm, r_vmem)
    pltpu.sync_copy(z_hbm.at[pl.ds(s * _RPT, _RPT), :],
                    y_acc.at[pl.ds(s * _RPT, _RPT), :])

    @pl.when(s == _NS - 1)
    def _():
        pltpu.sync_copy(z_hbm.at[pl.ds(_NS * _RPT, _TAIL), :],
                        y_acc.at[pl.ds(_NS * _RPT, _TAIL), :])

    plsc.subcore_barrier()

    A = (sidxA, didxA, xsA, excA, semXA, semEA)
    B = (sidxB, didxB, xsB, excB, semXB, semEB)

    def load_idx(j, buf):
        off = base + j * _CH
        pltpu.sync_copy(src_hbm.at[pl.ds(off, _CH)], buf[0])
        pltpu.sync_copy(dst_hbm.at[pl.ds(off, _CH)], buf[1])
        pltpu.async_copy(exg_hbm.at[pl.ds(off, _CH)], buf[3], buf[5])
        pltpu.async_copy(x_hbm.at[buf[0]], buf[2], buf[4])

    def wait_loads(j, buf):
        off = base + j * _CH
        pltpu.make_async_copy(exg_hbm.at[pl.ds(off, _CH)], buf[3],
                              buf[5]).wait()
        pltpu.make_async_copy(x_hbm.at[buf[0]], buf[2], buf[4]).wait()

    def half(j, cur, nxt, last):
        if not last:
            load_idx(j + 1, nxt)
        wait_loads(j, cur)
        sidx, didx, xs, exc = cur[0], cur[1], cur[2], cur[3]

        def gbody(g, carry2):
            d16 = didx[pl.ds(g * 16, 16)]
            rv = plsc.load_gather(r_vmem, [d16])
            adjc[pl.ds(g * 16, 16)] = exc[pl.ds(g * 16, 16)] * rv
            return carry2

        lax.fori_loop(0, _CH // 16, gbody, 0)
        _scale_rows(xs, exc)
        off = base + j * _CH
        w = pltpu.async_copy(adjc, adj_hbm.at[pl.ds(off, _CH)], semw)
        w.wait()
        w = pltpu.async_copy(xs, y_acc.at[didx], semw, add=True)
        w.wait()

    load_idx(0, A)

    def pair(k, carry):
        half(2 * k, A, B, False)
        half(2 * k + 1, B, A, False)
        return carry

    lax.fori_loop(0, (_NCHUNK - 1) // 2, pair, 0)
    half(_NCHUNK - 1, A, B, True)

    plsc.subcore_barrier()

    pltpu.sync_copy(y_acc.at[pl.ds(s * _RPT, _RPT), :],
                    yun_hbm.at[c, pl.ds(s * _RPT, _RPT), :])

    @pl.when(s == _NS - 1)
    def _():
        pltpu.sync_copy(y_acc.at[pl.ds(_NS * _RPT, _TAIL), :],
                        yun_hbm.at[c, pl.ds(_NS * _RPT, _TAIL), :])


def kernel(inputs, edge, W_gl, a_gl, W1, W2):
    src = edge[0].astype(_i32)
    dst = edge[1].astype(_i32)
    zeros = jnp.zeros((_N, _HID), _f32)
    zeros1 = jnp.zeros((_N,), _f32)

    h, x1 = pl.pallas_call(
        _mm_kernel,
        out_shape=[
            jax.ShapeDtypeStruct((_N, _HGL), _f32),
            jax.ShapeDtypeStruct((_N, _HID), _f32),
        ],
    )(inputs, W_gl, W1)
    hp = h.reshape(_N // 2, 2 * _HGL)

    edge_k = pl.kernel(
        _edge_kernel,
        out_type=[
            jax.ShapeDtypeStruct((_NC, _N, _HID), _f32),
            jax.ShapeDtypeStruct((_NC * _N,), _f32),
            jax.ShapeDtypeStruct((_E,), _f32),
        ],
        mesh=plsc.VectorSubcoreMesh(core_axis_name="c", subcore_axis_name="s"),
        compiler_params=pltpu.CompilerParams(needs_layout_passes=False),
        scratch_types=(
            2 * [
                pltpu.VMEM((_CH,), _i32),
                pltpu.VMEM((_CH,), _i32),
                pltpu.VMEM((_CH,), _i32),
                pltpu.VMEM((_CH,), _i32),
                pltpu.VMEM((_CH, _HID), _f32),
            ]
            + [
                pltpu.VMEM((_CH, 2 * _HGL), _f32),
                pltpu.VMEM((_CH, 2 * _HGL), _f32),
                pltpu.VMEM((_CH,), _f32),
                pltpu.VMEM((_HGL,), _f32),
                pltpu.VMEM((_RPT,), _f32),
                pltpu.VMEM_SHARED((_N, _HID), _f32),
                pltpu.VMEM_SHARED((_N,), _f32),
            ]
            + 6 * [pltpu.SemaphoreType.DMA]
            + 4 * [pltpu.SemaphoreType.DMA]
            + [pltpu.SemaphoreType.DMA]
        ),
    )
    y1un, s01, ex = edge_k(hp, x1, src, dst, a_gl, zeros, zeros1)

    s = s01[:_N] + s01[_N:]
    r = 1.0 / (s + 1e-16)
    r2d = r[:, None]

    x2 = pl.pallas_call(
        _mid_kernel,
        out_shape=jax.ShapeDtypeStruct((_N, _HID), _f32),
    )(y1un, r2d, W2)

    spmm_k = pl.kernel(
        _spmm_kernel,
        out_type=[
            jax.ShapeDtypeStruct((_NC, _N, _HID), _f32),
            jax.ShapeDtypeStruct((_E,), _f32),
        ],
        mesh=plsc.VectorSubcoreMesh(core_axis_name="c", subcore_axis_name="s"),
        compiler_params=pltpu.CompilerParams(needs_layout_passes=False),
        scratch_types=(
            2 * [
                pltpu.VMEM((_CH,), _i32),
                pltpu.VMEM((_CH,), _i32),
                pltpu.VMEM((_CH, _HID), _f32),
                pltpu.VMEM((_CH,), _f32),
            ]
            + [
                pltpu.VMEM((_CH,), _f32),
                pltpu.VMEM((_N,), _f32),
                pltpu.VMEM_SHARED((_N, _HID), _f32),
            ]
            + 4 * [pltpu.SemaphoreType.DMA]
            + [pltpu.SemaphoreType.DMA]
        ),
    )
    y2un, adj = spmm_k(x2, src, dst, ex, r, zeros)

    y2 = pl.pallas_call(
        _fin_kernel,
        out_shape=jax.ShapeDtypeStruct((_N, _HID), _f32),
    )(y2un, r2d)

    return (y2, h, adj)


# speedup vs baseline: 8.5625x; 1.0045x over previous
import jax
import jax.numpy as jnp
from jax import lax
from jax.experimental import pallas as pl
from jax.experimental.pallas import tpu as pltpu
from jax.experimental.pallas import tpu_sc as plsc

_N = 10000
_E = 320000
_HGL = 64
_HID = 128
_NC = 2
_NS = 16
_EPT = _E // (_NC * _NS)
_CH = 80
_NCHUNK = _EPT // _CH
_RPT = 624
_TAIL = _N - _NS * _RPT

_f32 = jnp.float32
_i32 = jnp.int32


def _mm_kernel(x_ref, wgl_ref, w1_ref, h_ref, x1_ref):
    x = x_ref[...]
    h_ref[...] = x @ wgl_ref[...]
    x1_ref[...] = x @ w1_ref[...]


def _mid_kernel(yun_ref, r_ref, w2_ref, x2_ref):
    y = yun_ref[0] + yun_ref[1]
    y = jnp.maximum(y * r_ref[...], 0.0)
    x2_ref[...] = y @ w2_ref[...]


def _fin_kernel(yun_ref, r_ref, out_ref):
    out_ref[...] = (yun_ref[0] + yun_ref[1]) * r_ref[...]


def _scale_rows(xs, exc):
    def gbody(g, carry):
        ev = exc[pl.ds(g * 16, 16)]
        for j in range(16):
            e = g * 16 + j
            w = ev[j]
            for k in range(_HID // 16):
                xs[e, pl.ds(k * 16, 16)] = xs[e, pl.ds(k * 16, 16)] * w
        return carry

    lax.fori_loop(0, _CH // 16, gbody, 0)


def _acc_init(z_hbm, z1_hbm, y_acc, s_acc, sb, s):
    pltpu.sync_copy(z_hbm.at[pl.ds(s * _RPT, _RPT), :],
                    y_acc.at[pl.ds(s * _RPT, _RPT), :])
    pltpu.sync_copy(z1_hbm.at[pl.ds(s * _RPT, _RPT)], sb)
    pltpu.sync_copy(sb, s_acc.at[pl.ds(s * _RPT, _RPT)])

    @pl.when(s == _NS - 1)
    def _():
        pltpu.sync_copy(z_hbm.at[pl.ds(_NS * _RPT, _TAIL), :],
                        y_acc.at[pl.ds(_NS * _RPT, _TAIL), :])
        pltpu.sync_copy(sb.at[pl.ds(0, _TAIL)],
                        s_acc.at[pl.ds(_NS * _RPT, _TAIL)])

    plsc.subcore_barrier()


def _edge_kernel(hp_hbm, x_hbm, src_hbm, dst_hbm, a_hbm, z_hbm, z1_hbm,
                 yun_hbm, s01_hbm, ex_hbm,
                 sidxA, didxA, s2A, d2A, xsA, didxSA, excSA,
                 sidxB, didxB, s2B, d2B, xsB, didxSB, excSB,
                 hs, hd, exc, a_buf, sb, y_acc, s_acc,
                 semXA, semSA, semDA, semXB, semSB, semDB,
                 semIsA, semIdA, semIsB, semIdB,
                 semW1, semW2, semW3):
    c = lax.axis_index("c")
    s = lax.axis_index("s")
    base = (c * _NS + s) * _EPT

    pltpu.sync_copy(a_hbm, a_buf)
    _acc_init(z_hbm, z1_hbm, y_acc, s_acc, sb, s)

    iota = lax.iota(_i32, 16)
    A = (sidxA, didxA, s2A, d2A, xsA, semXA, semSA, semDA, semIsA, semIdA,
         didxSA, excSA)
    B = (sidxB, didxB, s2B, d2B, xsB, semXB, semSB, semDB, semIsB, semIdB,
         didxSB, excSB)

    def issue_idx(j, buf):
        off = base + j * _CH
        pltpu.async_copy(src_hbm.at[pl.ds(off, _CH)], buf[0], buf[8])
        pltpu.async_copy(dst_hbm.at[pl.ds(off, _CH)], buf[1], buf[9])

    def wait_idx(j, buf):
        off = base + j * _CH
        pltpu.make_async_copy(src_hbm.at[pl.ds(off, _CH)], buf[0],
                              buf[8]).wait()
        pltpu.make_async_copy(dst_hbm.at[pl.ds(off, _CH)], buf[1],
                              buf[9]).wait()

    def shift_idx(buf):
        sidx, didx, s2, d2 = buf[0], buf[1], buf[2], buf[3]

        def gb(g, carry):
            s16 = sidx[pl.ds(g * 16, 16)]
            d16 = didx[pl.ds(g * 16, 16)]
            s2[pl.ds(g * 16, 16)] = jnp.right_shift(s16, 1)
            d2[pl.ds(g * 16, 16)] = jnp.right_shift(d16, 1)
            return carry

        lax.fori_loop(0, _CH // 16, gb, 0)

    def issue_x(buf):
        pltpu.async_copy(x_hbm.at[buf[0]], buf[4], buf[5])

    def wait_x(buf):
        pltpu.make_async_copy(x_hbm.at[buf[0]], buf[4], buf[5]).wait()

    def issue_h(buf):
        pltpu.async_copy(hp_hbm.at[buf[2]], hs, buf[6])
        pltpu.async_copy(hp_hbm.at[buf[3]], hd, buf[7])

    def wait_h(buf):
        pltpu.make_async_copy(hp_hbm.at[buf[2]], hs, buf[6]).wait()
        pltpu.make_async_copy(hp_hbm.at[buf[3]], hd, buf[7]).wait()

    def score(buf):
        sidx, didx = buf[0], buf[1]

        def gbody(g, carry):
            s16 = sidx[pl.ds(g * 16, 16)]
            d16 = didx[pl.ds(g * 16, 16)]
            ps = (s16 & 1) * _HGL
            pd = (d16 & 1) * _HGL
            rows = g * 16 + iota

            def fbody(fb, acc):
                av = a_buf[pl.ds(fb * 16, 16)]
                for j in range(16):
                    cs = ps + (fb * 16 + j)
                    cd = pd + (fb * 16 + j)
                    hsv = plsc.load_gather(hs, [rows, cs])
                    hdv = plsc.load_gather(hd, [rows, cd])
                    acc = acc + jnp.abs(hsv - hdv) * av[j]
                return acc

            acc = lax.fori_loop(0, _HGL // 16, fbody, jnp.zeros((16,), _f32))
            exc[pl.ds(g * 16, 16)] = jnp.exp(jnp.maximum(acc, 0.0))
            return carry

        lax.fori_loop(0, _CH // 16, gbody, 0)

    def wait_deferred(j, buf):
        didxS, excS = buf[10], buf[11]
        off = base + j * _CH
        pltpu.make_async_copy(excS, ex_hbm.at[pl.ds(off, _CH)],
                              semW1).wait()
        pltpu.make_async_copy(excS, s_acc.at[didxS], semW3).wait()

    def scatter(j, buf):
        didx, xs, didxS, excS = buf[1], buf[4], buf[10], buf[11]
        off = base + j * _CH

        def cpbody(g, carry):
            didxS[pl.ds(g * 16, 16)] = didx[pl.ds(g * 16, 16)]
            excS[pl.ds(g * 16, 16)] = exc[pl.ds(g * 16, 16)]
            return carry

        lax.fori_loop(0, _CH // 16, cpbody, 0)
        w = pltpu.async_copy(xs, y_acc.at[didx], semW2, add=True)
        w.wait()
        pltpu.async_copy(excS, ex_hbm.at[pl.ds(off, _CH)], semW1)
        pltpu.async_copy(excS, s_acc.at[didxS], semW3, add=True)

    def half(j, cur, nxt, last):
        if not last:
            wait_idx(j + 1, nxt)
            shift_idx(nxt)
            issue_x(nxt)
        wait_x(cur)
        wait_h(cur)
        score(cur)
        if not last:
            issue_h(nxt)
        _scale_rows(cur[4], exc)

        @pl.when(j > 0)
        def _():
            wait_deferred(j - 1, nxt)

        scatter(j, cur)
        if not last:
            @pl.when(j + 2 < _NCHUNK)
            def _():
                issue_idx(j + 2, cur)

    pltpu.sync_copy(src_hbm.at[pl.ds(base, _CH)], sidxA)
    pltpu.sync_copy(dst_hbm.at[pl.ds(base, _CH)], didxA)
    shift_idx(A)
    issue_x(A)
    issue_h(A)
    issue_idx(1, B)

    def pair(k, carry):
        half(2 * k, A, B, False)
        half(2 * k + 1, B, A, False)
        return carry

    lax.fori_loop(0, (_NCHUNK - 1) // 2, pair, 0)
    half(_NCHUNK - 1, A, B, True)
    wait_deferred(_NCHUNK - 1, A)

    plsc.subcore_barrier()

    pltpu.sync_copy(y_acc.at[pl.ds(s * _RPT, _RPT), :],
                    yun_hbm.at[c, pl.ds(s * _RPT, _RPT), :])
    pltpu.sync_copy(s_acc.at[pl.ds(s * _RPT, _RPT)], sb)
    pltpu.sync_copy(sb, s01_hbm.at[pl.ds(c * _N + s * _RPT, _RPT)])

    @pl.when(s == _NS - 1)
    def _():
        pltpu.sync_copy(y_acc.at[pl.ds(_NS * _RPT, _TAIL), :],
                        yun_hbm.at[c, pl.ds(_NS * _RPT, _TAIL), :])
        pltpu.sync_copy(s_acc.at[pl.ds(_NS * _RPT, _TAIL)],
                        sb.at[pl.ds(0, _TAIL)])
        pltpu.sync_copy(sb.at[pl.ds(0, _TAIL)],
                        s01_hbm.at[pl.ds(c * _N + _NS * _RPT, _TAIL)])


def _spmm_kernel(x_hbm, src_hbm, dst_hbm, exg_hbm, r_hbm, z_hbm,
                 yun_hbm, adj_hbm,
                 sidxA, didxA, xsA, excA,
                 sidxB, didxB, xsB, excB,
                 adjc, r_vmem, y_acc,
                 semXA, semEA, semXB, semEB, semw):
    c = lax.axis_index("c")
    s = lax.axis_index("s")
    base = (c * _NS + s) * _EPT

    pltpu.sync_copy(r_hbm, r_vmem)
    pltpu.sync_copy(z_hbm.at[pl.ds(s * _RPT, _RPT), :],
                    y_acc.at[pl.ds(s * _RPT, _RPT), :])

    @pl.when(s == _NS - 1)
    def _():
        pltpu.sync_copy(z_hbm.at[pl.ds(_NS * _RPT, _TAIL), :],
                        y_acc.at[pl.ds(_NS * _RPT, _TAIL), :])

    plsc.subcore_barrier()

    A = (sidxA, didxA, xsA, excA, semXA, semEA)
    B = (sidxB, didxB, xsB, excB, semXB, semEB)

    def load_idx(j, buf):
        off = base + j * _CH
        pltpu.sync_copy(src_hbm.at[pl.ds(off, _CH)], buf[0])
        pltpu.sync_copy(dst_hbm.at[pl.ds(off, _CH)], buf[1])
        pltpu.async_copy(exg_hbm.at[pl.ds(off, _CH)], buf[3], buf[5])
        pltpu.async_copy(x_hbm.at[buf[0]], buf[2], buf[4])

    def wait_loads(j, buf):
        off = base + j * _CH
        pltpu.make_async_copy(exg_hbm.at[pl.ds(off, _CH)], buf[3],
                              buf[5]).wait()
        pltpu.make_async_copy(x_hbm.at[buf[0]], buf[2], buf[4]).wait()

    def half(j, cur, nxt, last):
        if not last:
            load_idx(j + 1, nxt)
        wait_loads(j, cur)
        sidx, didx, xs, exc = cur[0], cur[1], cur[2], cur[3]

        def gbody(g, carry2):
            d16 = didx[pl.ds(g * 16, 16)]
            rv = plsc.load_gather(r_vmem, [d16])
            adjc[pl.ds(g * 16, 16)] = exc[pl.ds(g * 16, 16)] * rv
            return carry2

        lax.fori_loop(0, _CH // 16, gbody, 0)
        _scale_rows(xs, exc)
        off = base + j * _CH
        w = pltpu.async_copy(adjc, adj_hbm.at[pl.ds(off, _CH)], semw)
        w.wait()
        w = pltpu.async_copy(xs, y_acc.at[didx], semw, add=True)
        w.wait()

    load_idx(0, A)

    def pair(k, carry):
        half(2 * k, A, B, False)
        half(2 * k + 1, B, A, False)
        return carry

    lax.fori_loop(0, (_NCHUNK - 1) // 2, pair, 0)
    half(_NCHUNK - 1, A, B, True)

    plsc.subcore_barrier()

    pltpu.sync_copy(y_acc.at[pl.ds(s * _RPT, _RPT), :],
                    yun_hbm.at[c, pl.ds(s * _RPT, _RPT), :])

    @pl.when(s == _NS - 1)
    def _():
        pltpu.sync_copy(y_acc.at[pl.ds(_NS * _RPT, _TAIL), :],
                        yun_hbm.at[c, pl.ds(_NS * _RPT, _TAIL), :])


def kernel(inputs, edge, W_gl, a_gl, W1, W2):
    src = edge[0].astype(_i32)
    dst = edge[1].astype(_i32)
    zeros = jnp.zeros((_N, _HID), _f32)
    zeros1 = jnp.zeros((_N,), _f32)

    h, x1 = pl.pallas_call(
        _mm_kernel,
        out_shape=[
            jax.ShapeDtypeStruct((_N, _HGL), _f32),
            jax.ShapeDtypeStruct((_N, _HID), _f32),
        ],
    )(inputs, W_gl, W1)
    hp = h.reshape(_N // 2, 2 * _HGL)

    edge_k = pl.kernel(
        _edge_kernel,
        out_type=[
            jax.ShapeDtypeStruct((_NC, _N, _HID), _f32),
            jax.ShapeDtypeStruct((_NC * _N,), _f32),
            jax.ShapeDtypeStruct((_E,), _f32),
        ],
        mesh=plsc.VectorSubcoreMesh(core_axis_name="c", subcore_axis_name="s"),
        compiler_params=pltpu.CompilerParams(needs_layout_passes=False),
        scratch_types=(
            2 * [
                pltpu.VMEM((_CH,), _i32),
                pltpu.VMEM((_CH,), _i32),
                pltpu.VMEM((_CH,), _i32),
                pltpu.VMEM((_CH,), _i32),
                pltpu.VMEM((_CH, _HID), _f32),
                pltpu.VMEM((_CH,), _i32),
                pltpu.VMEM((_CH,), _f32),
            ]
            + [
                pltpu.VMEM((_CH, 2 * _HGL), _f32),
                pltpu.VMEM((_CH, 2 * _HGL), _f32),
                pltpu.VMEM((_CH,), _f32),
                pltpu.VMEM((_HGL,), _f32),
                pltpu.VMEM((_RPT,), _f32),
                pltpu.VMEM_SHARED((_N, _HID), _f32),
                pltpu.VMEM_SHARED((_N,), _f32),
            ]
            + 6 * [pltpu.SemaphoreType.DMA]
            + 4 * [pltpu.SemaphoreType.DMA]
            + 3 * [pltpu.SemaphoreType.DMA]
        ),
    )
    y1un, s01, ex = edge_k(hp, x1, src, dst, a_gl, zeros, zeros1)

    s = s01[:_N] + s01[_N:]
    r = 1.0 / (s + 1e-16)
    r2d = r[:, None]

    x2 = pl.pallas_call(
        _mid_kernel,
        out_shape=jax.ShapeDtypeStruct((_N, _HID), _f32),
    )(y1un, r2d, W2)

    spmm_k = pl.kernel(
        _spmm_kernel,
        out_type=[
            jax.ShapeDtypeStruct((_NC, _N, _HID), _f32),
            jax.ShapeDtypeStruct((_E,), _f32),
        ],
        mesh=plsc.VectorSubcoreMesh(core_axis_name="c", subcore_axis_name="s"),
        compiler_params=pltpu.CompilerParams(needs_layout_passes=False),
        scratch_types=(
            2 * [
                pltpu.VMEM((_CH,), _i32),
                pltpu.VMEM((_CH,), _i32),
                pltpu.VMEM((_CH, _HID), _f32),
                pltpu.VMEM((_CH,), _f32),
            ]
            + [
                pltpu.VMEM((_CH,), _f32),
                pltpu.VMEM((_N,), _f32),
                pltpu.VMEM_SHARED((_N, _HID), _f32),
            ]
            + 4 * [pltpu.SemaphoreType.DMA]
            + [pltpu.SemaphoreType.DMA]
        ),
    )
    y2un, adj = spmm_k(x2, src, dst, ex, r, zeros)

    y2 = pl.pallas_call(
        _fin_kernel,
        out_shape=jax.ShapeDtypeStruct((_N, _HID), _f32),
    )(y2un, r2d)

    return (y2, h, adj)
